# Initial kernel scaffold; baseline (speedup 1.0000x reference)
import jax
import jax.numpy as jnp
from jax.experimental import pallas as pl


def kernel(logits, labels, ins_T):
    raise NotImplementedError("write your pallas kernel here")


# speedup vs baseline: 11.3049x; 11.3049x over previous
import jax
import jax.numpy as jnp
from jax import lax
from jax.experimental import pallas as pl

_B, _C, _N = 8, 17, 2048
_K = 7
_RB = 512
_NRB = _N // _RB
_D_T = 289
_BIG = 3.0e38


def _loss_kernel(x_rows_ref, x_full_ref, lblr_ref, lblc_ref, t_ref, out_ref):
    b = pl.program_id(0)
    rb = pl.program_id(1)

    @pl.when(jnp.logical_and(b == 0, rb == 0))
    def _init():
        out_ref[...] = jnp.zeros_like(out_ref)

    xr = x_rows_ref[0].astype(jnp.float32)
    xf = x_full_ref[0].astype(jnp.float32)
    ones_c = jnp.ones((_C, 1), dtype=jnp.float32)

    s_col = lax.dot_general(xr * xr, ones_c, (((0,), (0,)), ((), ())),
                            preferred_element_type=jnp.float32)
    s_row = lax.dot_general(ones_c, xf * xf, (((0,), (0,)), ((), ())),
                            preferred_element_type=jnp.float32)
    g = lax.dot_general(xr, xf, (((0,), (0,)), ((), ())),
                        precision=lax.Precision.HIGHEST,
                        preferred_element_type=jnp.float32)
    d = s_col + s_row - 2.0 * g

    lbl_row = lblr_ref[0]
    lbl_col = lblc_ref[0]

    iota = lax.broadcasted_iota(jnp.int32, (_RB, _N), 1)

    w_sel = jnp.zeros((_RB, _N), dtype=jnp.float32)
    for t in range(_K + 1):
        m = jnp.min(d, axis=1, keepdims=True)
        sel = jnp.min(jnp.where(d == m, iota, _N), axis=1,
                      keepdims=True)
        onehot = iota == sel
        d = jnp.where(onehot, _BIG, d)
        if t >= 1:
            lbl_j = jnp.max(jnp.where(onehot, lbl_row, -1), axis=1,
                            keepdims=True)
            sign = jnp.where(lbl_j == lbl_col, 1.0, -1.0).astype(jnp.float32)
            w_t = sign * jnp.exp(-0.5 * m)
            w_sel = w_sel + jnp.where(onehot, w_t, 0.0)

    tb = t_ref[0]
    trows = t_ref[0, pl.ds(rb * _RB, _RB), :]
    ones_dt = jnp.ones((_D_T, 1), jnp.float32)
    p_full = lax.dot_general(tb * tb, ones_dt,
                             (((1,), (0,)), ((), ())),
                             preferred_element_type=jnp.float32)
    p_col = lax.dot_general(trows * trows, ones_dt,
                            (((1,), (0,)), ((), ())),
                            preferred_element_type=jnp.float32)

    a = jnp.sum(w_sel, axis=1, keepdims=True)
    term1 = jnp.sum(a * p_col)
    bcol = jnp.sum(w_sel, axis=0, keepdims=True)
    term2 = lax.dot_general(bcol, p_full, (((1,), (0,)), ((), ())),
                            preferred_element_type=jnp.float32)[0, 0]

    gmat = lax.dot_general(w_sel, tb, (((1,), (0,)), ((), ())),
                           precision=lax.Precision.HIGHEST,
                           preferred_element_type=jnp.float32)
    cross = jnp.sum(gmat * trows)

    partial = term1 + term2 - 2.0 * cross
    out_ref[...] += jnp.broadcast_to(partial, (1, 1))


def kernel(logits, labels, ins_T):
    x = logits
    lbl3r = labels.reshape(_B, 1, _N)
    lbl3c = labels.reshape(_B, _N, 1)
    t3 = ins_T.reshape(_B, _N, _D_T)

    total = pl.pallas_call(
        _loss_kernel,
        grid=(_B, _NRB),
        in_specs=[
            pl.BlockSpec((1, _C, _RB), lambda b, rb: (b, 0, rb)),
            pl.BlockSpec((1, _C, _N), lambda b, rb: (b, 0, 0)),
            pl.BlockSpec((1, 1, _N), lambda b, rb: (b, 0, 0)),
            pl.BlockSpec((1, _RB, 1), lambda b, rb: (b, rb, 0)),
            pl.BlockSpec((1, _N, _D_T), lambda b, rb: (b, 0, 0)),
        ],
        out_specs=pl.BlockSpec((1, 1), lambda b, rb: (0, 0)),
        out_shape=jax.ShapeDtypeStruct((1, 1), jnp.float32),
    )(x, x, lbl3r, lbl3c, t3)

    return total[0, 0] / jnp.float32(_B * _N * _K)


# speedup vs baseline: 23.5355x; 2.0819x over previous
import jax
import jax.numpy as jnp
from jax import lax
from jax.experimental import pallas as pl

_B, _C, _N = 8, 17, 2048
_K = 7
_RB = 512
_NRB = _N // _RB
_D_T = 289
_BIG = 3.0e38


def _dot3(a, b, dims):
    a_hi = a.astype(jnp.bfloat16).astype(jnp.float32)
    a_lo = a - a_hi
    b_hi = b.astype(jnp.bfloat16).astype(jnp.float32)
    b_lo = b - b_hi
    dd = (dims, ((), ()))
    out = lax.dot_general(a_hi, b_hi, dd, preferred_element_type=jnp.float32)
    out += lax.dot_general(a_hi, b_lo, dd, preferred_element_type=jnp.float32)
    out += lax.dot_general(a_lo, b_hi, dd, preferred_element_type=jnp.float32)
    return out


def _loss_kernel(x_rows_ref, x_full_ref, lblr_ref, lblc_ref, t_ref, out_ref):
    b = pl.program_id(0)
    rb = pl.program_id(1)

    @pl.when(jnp.logical_and(b == 0, rb == 0))
    def _init():
        out_ref[...] = jnp.zeros_like(out_ref)

    xr = x_rows_ref[0].astype(jnp.float32)
    xf = x_full_ref[0].astype(jnp.float32)
    ones_c = jnp.ones((_C, 1), dtype=jnp.float32)

    s_col = lax.dot_general(xr * xr, ones_c, (((0,), (0,)), ((), ())),
                            preferred_element_type=jnp.float32)
    s_row = lax.dot_general(ones_c, xf * xf, (((0,), (0,)), ((), ())),
                            preferred_element_type=jnp.float32)
    g = _dot3(xr, xf, ((0,), (0,)))
    d = s_col + s_row - 2.0 * g

    lbl_row = lblr_ref[0]
    lbl_col = lblc_ref[0]

    iota = lax.broadcasted_iota(jnp.int32, (_RB, _N), 1)

    bits = lax.bitcast_convert_type(jnp.maximum(d, 0.0), jnp.int32)
    key = (bits & jnp.int32(-2048)) | iota
    for _ in range(_K + 1):
        mkey = jnp.min(key, axis=1, keepdims=True)
        key = jnp.where(key == mkey, jnp.int32(0x7FFFFFFF), key)

    mask8 = key == jnp.int32(0x7FFFFFFF)
    sign = jnp.where(lbl_col == lbl_row, 1.0, -1.0).astype(jnp.float32)
    w_sel = jnp.where(mask8, sign * jnp.exp(-0.5 * d), 0.0)

    tb = t_ref[0]
    trows = t_ref[0, pl.ds(rb * _RB, _RB), :]
    ones_dt = jnp.ones((_D_T, 1), jnp.float32)
    p_full = lax.dot_general(tb * tb, ones_dt,
                             (((1,), (0,)), ((), ())),
                             preferred_element_type=jnp.float32)
    p_col = lax.dot_general(trows * trows, ones_dt,
                            (((1,), (0,)), ((), ())),
                            preferred_element_type=jnp.float32)

    a = jnp.sum(w_sel, axis=1, keepdims=True)
    term1 = jnp.sum(a * p_col)
    bcol = jnp.sum(w_sel, axis=0, keepdims=True)
    term2 = lax.dot_general(bcol, p_full, (((1,), (0,)), ((), ())),
                            preferred_element_type=jnp.float32)[0, 0]

    gmat = _dot3(w_sel, tb, ((1,), (0,)))
    cross = jnp.sum(gmat * trows)

    partial = term1 + term2 - 2.0 * cross
    out_ref[...] += jnp.broadcast_to(partial, (1, 1))


def kernel(logits, labels, ins_T):
    x = logits
    lbl3r = labels.reshape(_B, 1, _N)
    lbl3c = labels.reshape(_B, _N, 1)
    t3 = ins_T.reshape(_B, _N, _D_T)

    total = pl.pallas_call(
        _loss_kernel,
        grid=(_B, _NRB),
        in_specs=[
            pl.BlockSpec((1, _C, _RB), lambda b, rb: (b, 0, rb)),
            pl.BlockSpec((1, _C, _N), lambda b, rb: (b, 0, 0)),
            pl.BlockSpec((1, 1, _N), lambda b, rb: (b, 0, 0)),
            pl.BlockSpec((1, _RB, 1), lambda b, rb: (b, rb, 0)),
            pl.BlockSpec((1, _N, _D_T), lambda b, rb: (b, 0, 0)),
        ],
        out_specs=pl.BlockSpec((1, 1), lambda b, rb: (0, 0)),
        out_shape=jax.ShapeDtypeStruct((1, 1), jnp.float32),
    )(x, x, lbl3r, lbl3c, t3)

    return total[0, 0] / jnp.float32(_B * _N * _K)


# speedup vs baseline: 26.4630x; 1.1244x over previous
import jax
import jax.numpy as jnp
from jax import lax
from jax.experimental import pallas as pl

_B, _C, _N = 8, 17, 2048
_K = 7
_RB = 512
_NRB = _N // _RB
_D_T = 289
_BIG = 3.0e38


def _dot3(a, b, dims):
    a_hi = a.astype(jnp.bfloat16).astype(jnp.float32)
    a_lo = a - a_hi
    b_hi = b.astype(jnp.bfloat16).astype(jnp.float32)
    b_lo = b - b_hi
    dd = (dims, ((), ()))
    out = lax.dot_general(a_hi, b_hi, dd, preferred_element_type=jnp.float32)
    out += lax.dot_general(a_hi, b_lo, dd, preferred_element_type=jnp.float32)
    out += lax.dot_general(a_lo, b_hi, dd, preferred_element_type=jnp.float32)
    return out


def _loss_kernel(x_rows_ref, x_full_ref, lblr_ref, lblc_ref, t_ref, out_ref):
    b = pl.program_id(0)
    rb = pl.program_id(1)

    @pl.when(jnp.logical_and(b == 0, rb == 0))
    def _init():
        out_ref[...] = jnp.zeros_like(out_ref)

    xr = x_rows_ref[0].astype(jnp.float32)
    xf = x_full_ref[0].astype(jnp.float32)
    ones_c = jnp.ones((_C, 1), dtype=jnp.float32)

    s_col = lax.dot_general(xr * xr, ones_c, (((0,), (0,)), ((), ())),
                            preferred_element_type=jnp.float32)
    s_row = lax.dot_general(ones_c, xf * xf, (((0,), (0,)), ((), ())),
                            preferred_element_type=jnp.float32)
    g = _dot3(xr, xf, ((0,), (0,)))
    d = s_col + s_row - 2.0 * g

    lbl_row = lblr_ref[0]
    lbl_col = lblc_ref[0]

    iota = lax.broadcasted_iota(jnp.int32, (_RB, _N), 1)

    bits = lax.bitcast_convert_type(jnp.maximum(d, 0.0), jnp.int32)
    key = (bits & jnp.int32(-2048)) | iota
    mkey = jnp.min(key, axis=1, keepdims=True)
    onehot0 = key == mkey
    key = jnp.where(onehot0, jnp.int32(0x7FFFFFFF), key)
    for _ in range(_K):
        mkey = jnp.min(key, axis=1, keepdims=True)
        key = jnp.where(key == mkey, jnp.int32(0x7FFFFFFF), key)

    mask7 = (key == jnp.int32(0x7FFFFFFF)) & jnp.logical_not(onehot0)
    sign = jnp.where(lbl_col == lbl_row, 1.0, -1.0).astype(jnp.float32)
    w_sel = jnp.where(mask7, sign * jnp.exp(-0.5 * d), 0.0)

    tb = t_ref[0]
    trows = t_ref[0, pl.ds(rb * _RB, _RB), :]
    ones_dt = jnp.ones((_D_T, 1), jnp.float32)
    p_full = lax.dot_general(tb * tb, ones_dt,
                             (((1,), (0,)), ((), ())),
                             preferred_element_type=jnp.float32)
    p_col = lax.dot_general(trows * trows, ones_dt,
                            (((1,), (0,)), ((), ())),
                            preferred_element_type=jnp.float32)

    a = jnp.sum(w_sel, axis=1, keepdims=True)
    term1 = jnp.sum(a * p_col)
    bcol = jnp.sum(w_sel, axis=0, keepdims=True)
    term2 = lax.dot_general(bcol, p_full, (((1,), (0,)), ((), ())),
                            preferred_element_type=jnp.float32)[0, 0]

    gmat = lax.dot_general(w_sel, tb, (((1,), (0,)), ((), ())),
                           preferred_element_type=jnp.float32)
    cross = jnp.sum(gmat * trows)

    partial = term1 + term2 - 2.0 * cross
    out_ref[...] += jnp.broadcast_to(partial, (1, 1))


def kernel(logits, labels, ins_T):
    x = logits
    lbl3r = labels.reshape(_B, 1, _N)
    lbl3c = labels.reshape(_B, _N, 1)
    t3 = ins_T.reshape(_B, _N, _D_T)

    total = pl.pallas_call(
        _loss_kernel,
        grid=(_B, _NRB),
        in_specs=[
            pl.BlockSpec((1, _C, _RB), lambda b, rb: (b, 0, rb)),
            pl.BlockSpec((1, _C, _N), lambda b, rb: (b, 0, 0)),
            pl.BlockSpec((1, 1, _N), lambda b, rb: (b, 0, 0)),
            pl.BlockSpec((1, _RB, 1), lambda b, rb: (b, rb, 0)),
            pl.BlockSpec((1, _N, _D_T), lambda b, rb: (b, 0, 0)),
        ],
        out_specs=pl.BlockSpec((1, 1), lambda b, rb: (0, 0)),
        out_shape=jax.ShapeDtypeStruct((1, 1), jnp.float32),
    )(x, x, lbl3r, lbl3c, t3)

    return total[0, 0] / jnp.float32(_B * _N * _K)
